# Initial kernel scaffold; baseline (speedup 1.0000x reference)
import jax
import jax.numpy as jnp
from jax.experimental import pallas as pl


def kernel(x, edge_index, edge_attr, batch, emb_table, edge_W, edge_b, eps, W1, b1, g_in, be_in, W2, b2, g_out, be_out, fW1, fb1, fW2, fb2):
    raise NotImplementedError("write your pallas kernel here")


# speedup vs baseline: 5.0084x; 5.0084x over previous
import functools

import jax
import jax.numpy as jnp
from jax import lax
from jax.experimental import pallas as pl
from jax.experimental.pallas import tpu as pltpu
from jax.experimental.pallas import tpu_sc as plsc

N = 10000
E = 320000
D = 128
DE = 16
L = 4
B = 64
T = 10

_NC = 2
_NS = 16
_NW = _NC * _NS
_EPW = E // _NW
_CE = 40
_NT = _EPW // _CE
_RPS = 624
_RSTG = 48
_RREM = N - _NS * _RPS


def _edge_body(h_hbm, e_hbm, src_hbm, dst_hbm, agg_hbm,
               agg_sp, src_v, dst_v, ev_v, hm_v, stage_v,
               sem_i0, sem_i1, sem_i2, sem_i3, sem_e0, sem_e1,
               sem_g0, sem_g1):
    c = lax.axis_index("c")
    s = lax.axis_index("s")
    wid = c * _NS + s

    def _zrow(r, carry):
        for jb in range(D // 16):
            stage_v[r, pl.ds(jb * 16, 16)] = jnp.zeros((16,), jnp.float32)
        return carry
    lax.fori_loop(0, _RSTG, _zrow, 0)
    for t in range(_RPS // _RSTG):
        pltpu.sync_copy(stage_v, agg_sp.at[pl.ds(s * _RPS + t * _RSTG, _RSTG)])

    @pl.when(s == _NS - 1)
    def _():
        pltpu.sync_copy(stage_v.at[pl.ds(0, _RREM)],
                        agg_sp.at[pl.ds(_NS * _RPS, _RREM)])
    plsc.subcore_barrier()

    ebase = wid * _EPW
    sem_i = (sem_i0, sem_i1, sem_i2, sem_i3)
    sem_e = (sem_e0, sem_e1)
    sem_g = (sem_g0, sem_g1)

    def _issue_i(cc, slot):
        off = ebase + cc * _CE
        pltpu.async_copy(src_hbm.at[pl.ds(off, _CE)], src_v.at[slot],
                         sem_i[slot])
        pltpu.async_copy(dst_hbm.at[pl.ds(off, _CE)], dst_v.at[slot],
                         sem_i[slot])

    def _wait_i(slot):
        pltpu.make_async_copy(src_hbm.at[pl.ds(0, _CE)], src_v.at[slot],
                              sem_i[slot]).wait()
        pltpu.make_async_copy(dst_hbm.at[pl.ds(0, _CE)], dst_v.at[slot],
                              sem_i[slot]).wait()

    def _issue_e(cc, par):
        pltpu.async_copy(e_hbm.at[pl.ds(ebase + cc * _CE, _CE), :],
                         ev_v.at[par], sem_e[par])

    def _wait_e(par):
        pltpu.make_async_copy(e_hbm.at[pl.ds(0, _CE), :],
                              ev_v.at[par], sem_e[par]).wait()

    def _issue_g(slot, par):
        pltpu.async_copy(h_hbm.at[src_v.at[slot]], hm_v.at[par], sem_g[par])

    def _wait_g(par):
        pltpu.make_async_copy(h_hbm.at[src_v.at[0]],
                              hm_v.at[par], sem_g[par]).wait()

    def _compute(par):
        def ebody(e, carry):
            for jb in range(D // 16):
                v = (hm_v[par, e, pl.ds(jb * 16, 16)]
                     + ev_v[par, e, pl.ds(jb * 16, 16)])
                hm_v[par, e, pl.ds(jb * 16, 16)] = jnp.maximum(v, 0.0)
            return carry
        lax.fori_loop(0, _CE, ebody, 0)

    def _scatter(par, slot):
        pltpu.sync_copy(hm_v.at[par], agg_sp.at[dst_v.at[slot]], add=True)

    for cn in range(4):
        _issue_i(cn, cn)
    _wait_i(0)
    _issue_e(0, 0)
    _issue_e(1, 1)
    _issue_g(0, 0)

    def lbody(j, carry):
        for b in range(4):
            cc = j * 4 + b
            _wait_i((b + 1) % 4)
            _issue_g((b + 1) % 4, (b + 1) % 2)
            _wait_e(b % 2)
            _wait_g(b % 2)
            _compute(b % 2)
            _scatter(b % 2, b)

            @pl.when(cc + 4 < _NT)
            def _():
                _issue_i(cc + 4, b)
            _issue_e(cc + 2, b % 2)
        return carry
    lax.fori_loop(0, (_NT - 2) // 4, lbody, 0)

    _wait_i(1)
    _issue_g(1, 1)
    _wait_e(0)
    _wait_g(0)
    _compute(0)
    _scatter(0, 0)
    _wait_e(1)
    _wait_g(1)
    _compute(1)
    _scatter(1, 1)

    plsc.subcore_barrier()

    for t in range(_RPS // _RSTG):
        r0 = s * _RPS + t * _RSTG
        pltpu.sync_copy(agg_sp.at[pl.ds(r0, _RSTG)], stage_v)
        pltpu.sync_copy(stage_v, agg_hbm.at[c, pl.ds(r0, _RSTG)])

    @pl.when(s == _NS - 1)
    def _():
        r0 = _NS * _RPS
        pltpu.sync_copy(agg_sp.at[pl.ds(r0, _RREM)],
                        stage_v.at[pl.ds(0, _RREM)])
        pltpu.sync_copy(stage_v.at[pl.ds(0, _RREM)],
                        agg_hbm.at[c, pl.ds(r0, _RREM)])


@functools.cache
def _build_edge_call():
    return pl.kernel(
        _edge_body,
        out_type=jax.ShapeDtypeStruct((_NC, N, D), jnp.float32),
        mesh=plsc.VectorSubcoreMesh(core_axis_name="c", subcore_axis_name="s",
                                    num_cores=_NC, num_subcores=_NS),
        scratch_types=[
            pltpu.VMEM_SHARED((N, D), jnp.float32),
            pltpu.VMEM((4, _CE), jnp.int32),
            pltpu.VMEM((4, _CE), jnp.int32),
            pltpu.VMEM((2, _CE, D), jnp.float32),
            pltpu.VMEM((2, _CE, D), jnp.float32),
            pltpu.VMEM((_RSTG, D), jnp.float32),
        ] + [pltpu.SemaphoreType.DMA] * 8,
    )


def _embed_body(x_ref, emb_ref, out_ref):
    oh = (x_ref[:] == lax.broadcasted_iota(jnp.int32, (N, 32), 1))
    out_ref[:] = jnp.dot(oh.astype(jnp.float32), emb_ref[:],
                         preferred_element_type=jnp.float32)


def _ematmul_body(ea_ref, w_ref, b_ref, out_ref):
    out_ref[:] = (jnp.dot(ea_ref[:], w_ref[:],
                          preferred_element_type=jnp.float32) + b_ref[:])


_BE = 8000


def _node_update(h, agg, w1, b1, gin, bein, w2, b2, gout, beout, eps1p):
    z = eps1p * h + agg[0] + agg[1]
    z = jnp.dot(z, w1, preferred_element_type=jnp.float32) + b1
    mu = jnp.mean(z, axis=0, keepdims=True)
    var = jnp.mean((z - mu) ** 2, axis=0, keepdims=True)
    z = jnp.maximum((z - mu) * lax.rsqrt(var + 1e-5) * gin + bein, 0.0)
    z = jnp.dot(z, w2, preferred_element_type=jnp.float32) + b2
    mu2 = jnp.mean(z, axis=0, keepdims=True)
    var2 = jnp.mean((z - mu2) ** 2, axis=0, keepdims=True)
    return jnp.maximum((z - mu2) * lax.rsqrt(var2 + 1e-5) * gout + beout, 0.0)


def _node_body(h_ref, agg_ref, w1_ref, b1_ref, gin_ref, bein_ref,
               w2_ref, b2_ref, gout_ref, beout_ref, eps_ref, out_ref):
    out_ref[:] = _node_update(
        h_ref[:], agg_ref, w1_ref[:], b1_ref[:], gin_ref[:], bein_ref[:],
        w2_ref[:], b2_ref[:], gout_ref[:], beout_ref[:], eps_ref[:])


def _final_body(h_ref, agg_ref, w1_ref, b1_ref, gin_ref, bein_ref,
                w2_ref, b2_ref, gout_ref, beout_ref, eps_ref,
                batch_ref, fw1_ref, fb1_ref, fw2_ref, fb2_ref, out_ref):
    hout = _node_update(
        h_ref[:], agg_ref, w1_ref[:], b1_ref[:], gin_ref[:], bein_ref[:],
        w2_ref[:], b2_ref[:], gout_ref[:], beout_ref[:], eps_ref[:])
    oh = (batch_ref[:] == lax.broadcasted_iota(jnp.int32, (B, N), 0))
    pooled = jnp.dot(oh.astype(jnp.float32), hout,
                     preferred_element_type=jnp.float32)
    t = jnp.maximum(jnp.dot(pooled, fw1_ref[:],
                            preferred_element_type=jnp.float32) + fb1_ref[:],
                    0.0)
    out_ref[:] = (jnp.dot(t, fw2_ref[:],
                          preferred_element_type=jnp.float32) + fb2_ref[:])


def kernel(x, edge_index, edge_attr, batch, emb_table, edge_W, edge_b, eps,
           W1, b1, g_in, be_in, W2, b2, g_out, be_out, fW1, fb1, fW2, fb2):
    f32 = jnp.float32

    emb_pad = jnp.zeros((32, D), f32).at[:21].set(emb_table)
    x2 = x.astype(jnp.int32).reshape(N, 1)
    src1 = edge_index[0].astype(jnp.int32)
    dst1 = edge_index[1].astype(jnp.int32)
    batch2 = batch.astype(jnp.int32).reshape(1, N)

    h = pl.pallas_call(
        _embed_body,
        out_shape=jax.ShapeDtypeStruct((N, D), f32),
    )(x2, emb_pad)

    ematmul = pl.pallas_call(
        _ematmul_body,
        grid=(E // _BE,),
        in_specs=[
            pl.BlockSpec((_BE, DE), lambda i: (i, 0)),
            pl.BlockSpec((DE, D), lambda i: (0, 0)),
            pl.BlockSpec((1, D), lambda i: (0, 0)),
        ],
        out_specs=pl.BlockSpec((_BE, D), lambda i: (i, 0)),
        out_shape=jax.ShapeDtypeStruct((E, D), f32),
    )

    for i in range(L):
        e_i = ematmul(edge_attr, edge_W[i], edge_b[i].reshape(1, D))
        agg = _build_edge_call()(h, e_i, src1, dst1)
        layer_args = (
            h, agg, W1[i], b1[i].reshape(1, D), g_in[i].reshape(1, D),
            be_in[i].reshape(1, D), W2[i], b2[i].reshape(1, D),
            g_out[i].reshape(1, D), be_out[i].reshape(1, D),
            (1.0 + eps[i]).astype(f32).reshape(1, 1),
        )
        if i < L - 1:
            h = pl.pallas_call(
                _node_body,
                out_shape=jax.ShapeDtypeStruct((N, D), f32),
            )(*layer_args)
        else:
            out = pl.pallas_call(
                _final_body,
                out_shape=jax.ShapeDtypeStruct((B, T), f32),
            )(*layer_args, batch2, fW1, fb1.reshape(1, 2 * D),
              fW2, fb2.reshape(1, T))
    return out


# speedup vs baseline: 5.3251x; 1.0632x over previous
import functools

import jax
import jax.numpy as jnp
from jax import lax
from jax.experimental import pallas as pl
from jax.experimental.pallas import tpu as pltpu
from jax.experimental.pallas import tpu_sc as plsc

N = 10000
E = 320000
D = 128
DE = 16
L = 4
B = 64
T = 10

_NC = 2
_NS = 16
_NW = _NC * _NS
_EPW = E // _NW
_CE = 40
_NT = _EPW // _CE
_RPS = 624
_RSTG = 48
_RREM = N - _NS * _RPS


def _edge_body(h_hbm, e_hbm, src_hbm, dst_hbm, agg_hbm,
               agg_sp, src_v, dst_v, ev_v, hm_v, stage_v,
               sem_i0, sem_i1, sem_i2, sem_i3, sem_e0, sem_e1,
               sem_g0, sem_g1, sem_g2, sem_g3,
               sem_s0, sem_s1, sem_s2, sem_s3):
    c = lax.axis_index("c")
    s = lax.axis_index("s")
    wid = c * _NS + s

    def _zrow(r, carry):
        for jb in range(D // 16):
            stage_v[r, pl.ds(jb * 16, 16)] = jnp.zeros((16,), jnp.float32)
        return carry
    lax.fori_loop(0, _RSTG, _zrow, 0)
    for t in range(_RPS // _RSTG):
        pltpu.sync_copy(stage_v, agg_sp.at[pl.ds(s * _RPS + t * _RSTG, _RSTG)])

    @pl.when(s == _NS - 1)
    def _():
        pltpu.sync_copy(stage_v.at[pl.ds(0, _RREM)],
                        agg_sp.at[pl.ds(_NS * _RPS, _RREM)])
    plsc.subcore_barrier()

    ebase = wid * _EPW
    sem_i = (sem_i0, sem_i1, sem_i2, sem_i3)
    sem_e = (sem_e0, sem_e1)
    sem_g = (sem_g0, sem_g1, sem_g2, sem_g3)
    sem_s = (sem_s0, sem_s1, sem_s2, sem_s3)

    def _issue_i(cc, slot):
        off = ebase + cc * _CE
        pltpu.async_copy(src_hbm.at[pl.ds(off, _CE)], src_v.at[slot],
                         sem_i[slot])
        pltpu.async_copy(dst_hbm.at[pl.ds(off, _CE)], dst_v.at[slot],
                         sem_i[slot])

    def _wait_i(slot):
        pltpu.make_async_copy(src_hbm.at[pl.ds(0, _CE)], src_v.at[slot],
                              sem_i[slot]).wait()
        pltpu.make_async_copy(dst_hbm.at[pl.ds(0, _CE)], dst_v.at[slot],
                              sem_i[slot]).wait()

    def _issue_e(cc, par):
        pltpu.async_copy(e_hbm.at[pl.ds(ebase + cc * _CE, _CE), :],
                         ev_v.at[par], sem_e[par])

    def _wait_e(par):
        pltpu.make_async_copy(e_hbm.at[pl.ds(0, _CE), :],
                              ev_v.at[par], sem_e[par]).wait()

    def _issue_g(slot):
        pltpu.async_copy(h_hbm.at[src_v.at[slot]], hm_v.at[slot], sem_g[slot])

    def _wait_g(slot):
        pltpu.make_async_copy(h_hbm.at[src_v.at[0]],
                              hm_v.at[slot], sem_g[slot]).wait()

    _EB = 4

    def _compute(slot, par_e):
        def ebody(g, carry):
            for eb in range(_EB):
                e = g * _EB + eb
                for jb in range(D // 16):
                    v = (hm_v[slot, e, pl.ds(jb * 16, 16)]
                         + ev_v[par_e, e, pl.ds(jb * 16, 16)])
                    hm_v[slot, e, pl.ds(jb * 16, 16)] = jnp.maximum(v, 0.0)
            return carry
        lax.fori_loop(0, _CE // _EB, ebody, 0)

    def _issue_s(slot):
        pltpu.async_copy(hm_v.at[slot], agg_sp.at[dst_v.at[slot]],
                         sem_s[slot], add=True)

    def _wait_s(slot):
        pltpu.make_async_copy(hm_v.at[slot], agg_sp.at[dst_v.at[slot]],
                              sem_s[slot]).wait()

    _issue_i(0, 0)
    _issue_i(1, 1)
    _wait_i(0)
    _issue_e(0, 0)
    _issue_e(1, 1)
    _issue_g(0)

    def lbody(j, carry):
        for b in range(4):
            cc = j * 4 + b

            @pl.when(cc >= 2)
            def _():
                _wait_s((b + 2) % 4)
            _issue_i(cc + 2, (b + 2) % 4)
            _wait_i((b + 1) % 4)
            _issue_g((b + 1) % 4)
            _wait_e(b % 2)
            _wait_g(b)
            _compute(b, b % 2)
            _issue_s(b)
            _issue_e(cc + 2, b % 2)
        return carry
    lax.fori_loop(0, (_NT - 2) // 4, lbody, 0)

    _wait_s(2)
    _wait_i(1)
    _issue_g(1)
    _wait_e(0)
    _wait_g(0)
    _compute(0, 0)
    _issue_s(0)
    _wait_s(3)
    _wait_e(1)
    _wait_g(1)
    _compute(1, 1)
    _issue_s(1)
    _wait_s(0)
    _wait_s(1)

    plsc.subcore_barrier()

    for t in range(_RPS // _RSTG):
        r0 = s * _RPS + t * _RSTG
        pltpu.sync_copy(agg_sp.at[pl.ds(r0, _RSTG)], stage_v)
        pltpu.sync_copy(stage_v, agg_hbm.at[c, pl.ds(r0, _RSTG)])

    @pl.when(s == _NS - 1)
    def _():
        r0 = _NS * _RPS
        pltpu.sync_copy(agg_sp.at[pl.ds(r0, _RREM)],
                        stage_v.at[pl.ds(0, _RREM)])
        pltpu.sync_copy(stage_v.at[pl.ds(0, _RREM)],
                        agg_hbm.at[c, pl.ds(r0, _RREM)])


@functools.cache
def _build_edge_call():
    return pl.kernel(
        _edge_body,
        out_type=jax.ShapeDtypeStruct((_NC, N, D), jnp.float32),
        mesh=plsc.VectorSubcoreMesh(core_axis_name="c", subcore_axis_name="s",
                                    num_cores=_NC, num_subcores=_NS),
        scratch_types=[
            pltpu.VMEM_SHARED((N, D), jnp.float32),
            pltpu.VMEM((4, _CE), jnp.int32),
            pltpu.VMEM((4, _CE), jnp.int32),
            pltpu.VMEM((2, _CE, D), jnp.float32),
            pltpu.VMEM((4, _CE, D), jnp.float32),
            pltpu.VMEM((_RSTG, D), jnp.float32),
        ] + [pltpu.SemaphoreType.DMA] * 14,
    )


def _embed_body(x_ref, emb_ref, out_ref):
    oh = (x_ref[:] == lax.broadcasted_iota(jnp.int32, (N, 32), 1))
    out_ref[:] = jnp.dot(oh.astype(jnp.float32), emb_ref[:],
                         preferred_element_type=jnp.float32)


def _ematmul_body(ea_ref, w_ref, b_ref, out_ref):
    out_ref[:] = (jnp.dot(ea_ref[:], w_ref[:],
                          preferred_element_type=jnp.float32) + b_ref[:])


_BE = 8000


def _node_update(h, agg, w1, b1, gin, bein, w2, b2, gout, beout, eps1p):
    z = eps1p * h + agg[0] + agg[1]
    z = jnp.dot(z, w1, preferred_element_type=jnp.float32) + b1
    mu = jnp.mean(z, axis=0, keepdims=True)
    var = jnp.mean((z - mu) ** 2, axis=0, keepdims=True)
    z = jnp.maximum((z - mu) * lax.rsqrt(var + 1e-5) * gin + bein, 0.0)
    z = jnp.dot(z, w2, preferred_element_type=jnp.float32) + b2
    mu2 = jnp.mean(z, axis=0, keepdims=True)
    var2 = jnp.mean((z - mu2) ** 2, axis=0, keepdims=True)
    return jnp.maximum((z - mu2) * lax.rsqrt(var2 + 1e-5) * gout + beout, 0.0)


def _node_body(h_ref, agg_ref, w1_ref, b1_ref, gin_ref, bein_ref,
               w2_ref, b2_ref, gout_ref, beout_ref, eps_ref, out_ref):
    out_ref[:] = _node_update(
        h_ref[:], agg_ref, w1_ref[:], b1_ref[:], gin_ref[:], bein_ref[:],
        w2_ref[:], b2_ref[:], gout_ref[:], beout_ref[:], eps_ref[:])


def _final_body(h_ref, agg_ref, w1_ref, b1_ref, gin_ref, bein_ref,
                w2_ref, b2_ref, gout_ref, beout_ref, eps_ref,
                batch_ref, fw1_ref, fb1_ref, fw2_ref, fb2_ref, out_ref):
    hout = _node_update(
        h_ref[:], agg_ref, w1_ref[:], b1_ref[:], gin_ref[:], bein_ref[:],
        w2_ref[:], b2_ref[:], gout_ref[:], beout_ref[:], eps_ref[:])
    oh = (batch_ref[:] == lax.broadcasted_iota(jnp.int32, (B, N), 0))
    pooled = jnp.dot(oh.astype(jnp.float32), hout,
                     preferred_element_type=jnp.float32)
    t = jnp.maximum(jnp.dot(pooled, fw1_ref[:],
                            preferred_element_type=jnp.float32) + fb1_ref[:],
                    0.0)
    out_ref[:] = (jnp.dot(t, fw2_ref[:],
                          preferred_element_type=jnp.float32) + fb2_ref[:])


def kernel(x, edge_index, edge_attr, batch, emb_table, edge_W, edge_b, eps,
           W1, b1, g_in, be_in, W2, b2, g_out, be_out, fW1, fb1, fW2, fb2):
    f32 = jnp.float32

    emb_pad = jnp.zeros((32, D), f32).at[:21].set(emb_table)
    x2 = x.astype(jnp.int32).reshape(N, 1)
    src1 = edge_index[0].astype(jnp.int32)
    dst1 = edge_index[1].astype(jnp.int32)
    batch2 = batch.astype(jnp.int32).reshape(1, N)

    h = pl.pallas_call(
        _embed_body,
        out_shape=jax.ShapeDtypeStruct((N, D), f32),
    )(x2, emb_pad)

    ematmul = pl.pallas_call(
        _ematmul_body,
        grid=(E // _BE,),
        in_specs=[
            pl.BlockSpec((_BE, DE), lambda i: (i, 0)),
            pl.BlockSpec((DE, D), lambda i: (0, 0)),
            pl.BlockSpec((1, D), lambda i: (0, 0)),
        ],
        out_specs=pl.BlockSpec((_BE, D), lambda i: (i, 0)),
        out_shape=jax.ShapeDtypeStruct((E, D), f32),
    )

    for i in range(L):
        e_i = ematmul(edge_attr, edge_W[i], edge_b[i].reshape(1, D))
        agg = _build_edge_call()(h, e_i, src1, dst1)
        layer_args = (
            h, agg, W1[i], b1[i].reshape(1, D), g_in[i].reshape(1, D),
            be_in[i].reshape(1, D), W2[i], b2[i].reshape(1, D),
            g_out[i].reshape(1, D), be_out[i].reshape(1, D),
            (1.0 + eps[i]).astype(f32).reshape(1, 1),
        )
        if i < L - 1:
            h = pl.pallas_call(
                _node_body,
                out_shape=jax.ShapeDtypeStruct((N, D), f32),
            )(*layer_args)
        else:
            out = pl.pallas_call(
                _final_body,
                out_shape=jax.ShapeDtypeStruct((B, T), f32),
            )(*layer_args, batch2, fW1, fb1.reshape(1, 2 * D),
              fW2, fb2.reshape(1, T))
    return out
